# Initial kernel scaffold; baseline (speedup 1.0000x reference)
import functools

import jax
import jax.numpy as jnp
from jax import lax
from jax.experimental import pallas as pl
from jax.experimental.pallas import tpu as pltpu
from jax.experimental.pallas import tpu_sc as plsc

ROWS_PER_STREAM = 128
K_STREAMS = 8
CHUNK = K_STREAMS * ROWS_PER_STREAM


@functools.partial(jax.jit, static_argnums=(2, 3, 4))
def _sc_gather(idx2d, table, n_rows, nc, ns):
    d = table.shape[1]
    nw = nc * ns
    rows_per_w = n_rows // nw
    chunks_per_w = rows_per_w // CHUNK
    mesh = plsc.VectorSubcoreMesh(
        core_axis_name="c", subcore_axis_name="s",
        num_cores=nc, num_subcores=ns)

    @functools.partial(
        pl.kernel,
        out_type=jax.ShapeDtypeStruct((n_rows, d), jnp.float32),
        mesh=mesh,
        scratch_types=[
            pltpu.VMEM((K_STREAMS, ROWS_PER_STREAM), jnp.int32),
            pltpu.VMEM((CHUNK, d), jnp.float32),
            pltpu.SemaphoreType.DMA,
        ],
    )
    def k(table_hbm, idx_hbm, out_hbm, idx_v, rows_v, sem):
        wid = lax.axis_index("s") * nc + lax.axis_index("c")
        base_row = wid * rows_per_w

        def chunk_body(g, _):
            row0 = base_row + g * CHUNK
            pltpu.sync_copy(idx_hbm.at[pl.ds(row0 // ROWS_PER_STREAM,
                                             K_STREAMS)], idx_v)
            copies = [
                pltpu.async_copy(
                    table_hbm.at[idx_v.at[j]],
                    rows_v.at[pl.ds(j * ROWS_PER_STREAM, ROWS_PER_STREAM)],
                    sem)
                for j in range(K_STREAMS)
            ]
            for c in copies:
                c.wait()
            pltpu.sync_copy(rows_v, out_hbm.at[pl.ds(row0, CHUNK)])
            return 0

        lax.fori_loop(0, chunks_per_w, chunk_body, 0)

    return k(table, idx2d)


def kernel(input_indices, table):
    b, f = input_indices.shape
    v, d = table.shape
    n_rows = b * f
    idx2d = input_indices.reshape(n_rows // ROWS_PER_STREAM,
                                  ROWS_PER_STREAM).astype(jnp.int32)
    info = plsc.get_sparse_core_info()
    out = _sc_gather(idx2d, table, n_rows, info.num_cores, info.num_subcores)
    return out.reshape(b, f, d)


# speedup vs baseline: 1.5471x; 1.5471x over previous
import functools

import jax
import jax.numpy as jnp
from jax import lax
from jax.experimental import pallas as pl
from jax.experimental.pallas import tpu as pltpu
from jax.experimental.pallas import tpu_sc as plsc

ROWS_PER_STREAM = 128
K_STREAMS = 8
CHUNK = K_STREAMS * ROWS_PER_STREAM


@functools.partial(jax.jit, static_argnums=(2, 3, 4))
def _sc_gather(idx2d, table, n_rows, nc, ns):
    d = table.shape[1]
    nw = nc * ns
    rows_per_w = n_rows // nw
    chunks_per_w = rows_per_w // CHUNK
    mesh = plsc.VectorSubcoreMesh(
        core_axis_name="c", subcore_axis_name="s",
        num_cores=nc, num_subcores=ns)

    @functools.partial(
        pl.kernel,
        out_type=jax.ShapeDtypeStruct((n_rows, d), jnp.float32),
        mesh=mesh,
        scratch_types=[
            pltpu.VMEM((K_STREAMS, ROWS_PER_STREAM), jnp.int32),
            pltpu.VMEM((CHUNK, d), jnp.float32),
            pltpu.SemaphoreType.DMA,
        ],
        compiler_params=pltpu.CompilerParams(use_tc_tiling_on_sc=False),
    )
    def k(table_hbm, idx_hbm, out_hbm, idx_v, rows_v, sem):
        wid = lax.axis_index("s") * nc + lax.axis_index("c")
        base_row = wid * rows_per_w

        def chunk_body(g, _):
            row0 = pl.multiple_of(base_row + g * CHUNK, CHUNK)
            idx_row0 = pl.multiple_of(
                (base_row // ROWS_PER_STREAM) + g * K_STREAMS, K_STREAMS)
            pltpu.sync_copy(idx_hbm.at[pl.ds(idx_row0, K_STREAMS)], idx_v)
            copies = [
                pltpu.async_copy(
                    table_hbm.at[idx_v.at[j]],
                    rows_v.at[pl.ds(j * ROWS_PER_STREAM, ROWS_PER_STREAM)],
                    sem)
                for j in range(K_STREAMS)
            ]
            for c in copies:
                c.wait()
            pltpu.sync_copy(rows_v, out_hbm.at[pl.ds(row0, CHUNK)])
            return 0

        lax.fori_loop(0, chunks_per_w, chunk_body, 0)

    return k(table, idx2d)


def kernel(input_indices, table):
    b, f = input_indices.shape
    v, d = table.shape
    n_rows = b * f
    idx2d = input_indices.reshape(n_rows // ROWS_PER_STREAM,
                                  ROWS_PER_STREAM).astype(jnp.int32)
    info = plsc.get_sparse_core_info()
    out = _sc_gather(idx2d, table, n_rows, info.num_cores, info.num_subcores)
    return out.reshape(b, f, d)


# speedup vs baseline: 1.5756x; 1.0185x over previous
import functools

import jax
import jax.numpy as jnp
from jax import lax
from jax.experimental import pallas as pl
from jax.experimental.pallas import tpu as pltpu
from jax.experimental.pallas import tpu_sc as plsc

ROWS_PER_STREAM = 128
K_STREAMS = 8
CHUNK = K_STREAMS * ROWS_PER_STREAM


@functools.partial(jax.jit, static_argnums=(2, 3, 4))
def _sc_gather(idx2d, table, n_rows, nc, ns):
    d = table.shape[1]
    nw = nc * ns
    rows_per_w = n_rows // nw
    nch = rows_per_w // CHUNK
    idx_rows_w = rows_per_w // ROWS_PER_STREAM
    mesh = plsc.VectorSubcoreMesh(
        core_axis_name="c", subcore_axis_name="s",
        num_cores=nc, num_subcores=ns)

    @functools.partial(
        pl.kernel,
        out_type=jax.ShapeDtypeStruct((n_rows, d), jnp.float32),
        mesh=mesh,
        scratch_types=[
            pltpu.VMEM((idx_rows_w, ROWS_PER_STREAM), jnp.int32),
            pltpu.VMEM((CHUNK, d), jnp.float32),
            pltpu.VMEM((CHUNK, d), jnp.float32),
            pltpu.SemaphoreType.DMA,
            pltpu.SemaphoreType.DMA,
            pltpu.SemaphoreType.DMA,
            pltpu.SemaphoreType.DMA,
        ],
        compiler_params=pltpu.CompilerParams(use_tc_tiling_on_sc=False),
    )
    def k(table_hbm, idx_hbm, out_hbm, idx_v, rows0, rows1,
          sg0, sg1, so0, so1):
        wid = lax.axis_index("s") * nc + lax.axis_index("c")
        base_row = wid * rows_per_w
        rows = (rows0, rows1)
        sg = (sg0, sg1)
        so = (so0, so1)

        idx_base = pl.multiple_of(wid * idx_rows_w, 8)
        pltpu.sync_copy(idx_hbm.at[pl.ds(idx_base, idx_rows_w)], idx_v)

        def issue(c, b):
            for j in range(K_STREAMS):
                pltpu.async_copy(
                    table_hbm.at[idx_v.at[c * K_STREAMS + j]],
                    rows[b].at[pl.ds(j * ROWS_PER_STREAM, ROWS_PER_STREAM)],
                    sg[b])

        def consume(c, b):
            for j in range(K_STREAMS):
                pltpu.make_async_copy(
                    table_hbm.at[idx_v.at[j]],
                    rows[b].at[pl.ds(j * ROWS_PER_STREAM, ROWS_PER_STREAM)],
                    sg[b]).wait()
            row0 = pl.multiple_of(base_row + c * CHUNK, CHUNK)
            pltpu.async_copy(rows[b], out_hbm.at[pl.ds(row0, CHUNK)], so[b])

        def wait_out(b):
            pltpu.make_async_copy(
                rows[b], out_hbm.at[pl.ds(0, CHUNK)], so[b]).wait()

        issue(0, 0)
        issue(1, 1)

        def body(i, _):
            c0 = 2 * i
            consume(c0, 0)

            @pl.when(c0 + 2 < nch)
            def _():
                wait_out(0)
                issue(c0 + 2, 0)

            consume(c0 + 1, 1)

            @pl.when(c0 + 3 < nch)
            def _():
                wait_out(1)
                issue(c0 + 3, 1)

            return 0

        if nch % 2:
            lax.fori_loop(0, (nch - 1) // 2, body, 0)
            consume(nch - 1, 0)
        else:
            lax.fori_loop(0, nch // 2 - 1, body, 0)
            consume(nch - 2, 0)
            consume(nch - 1, 1)
        wait_out(0)
        wait_out(1)

    return k(table, idx2d)


def kernel(input_indices, table):
    b, f = input_indices.shape
    v, d = table.shape
    n_rows = b * f
    idx2d = input_indices.reshape(n_rows // ROWS_PER_STREAM,
                                  ROWS_PER_STREAM).astype(jnp.int32)
    info = plsc.get_sparse_core_info()
    out = _sc_gather(idx2d, table, n_rows, info.num_cores, info.num_subcores)
    return out.reshape(b, f, d)
